# baseline (speedup 1.0000x reference)
import functools
import jax
import jax.numpy as jnp
from jax import lax
from jax.experimental import pallas as pl
from jax.experimental.pallas import tpu as pltpu
from jax.experimental.pallas import tpu_sc as plsc

_L = 16
_GROUP = 16


def _sc_mix(N, D, NW):
    rpw = N // NW
    ngrp = rpw // _GROUP
    gelems = _GROUP * D

    mesh = plsc.VectorSubcoreMesh(core_axis_name="c", subcore_axis_name="s")

    @functools.partial(
        pl.kernel,
        out_type=jax.ShapeDtypeStruct((N * D,), jnp.float32),
        mesh=mesh,
        scratch_types=[
            pltpu.VMEM((gelems,), jnp.float32),
            pltpu.VMEM((gelems,), jnp.float32),
            pltpu.VMEM((rpw * _L,), jnp.float32),
            pltpu.VMEM((D,), jnp.float32),
            pltpu.VMEM((D,), jnp.float32),
            pltpu.VMEM((_L,), jnp.float32),
            pltpu.VMEM((_L,), jnp.float32),
            pltpu.VMEM((_L,), jnp.float32),
        ],
    )
    def k(tok_hbm, pos_hbm, mask_hbm, me_hbm, tw_hbm, pw_hbm, mw_hbm,
          out_hbm, tok_buf, pos_buf, mask_buf, me_buf, mrow_buf,
          tw_buf, pw_buf, mw_buf):
        wid = lax.axis_index("s") * 2 + lax.axis_index("c")
        base = wid * rpw

        pltpu.sync_copy(tw_hbm, tw_buf)
        pltpu.sync_copy(pw_hbm, pw_buf)
        pltpu.sync_copy(mw_hbm, mw_buf)
        pltpu.sync_copy(me_hbm, me_buf)
        pltpu.sync_copy(mask_hbm.at[pl.ds(base * _L, rpw * _L)], mask_buf)

        twv = tw_buf[...]
        pwv = pw_buf[...]
        mwv = mw_buf[...]

        def scale_me(kk, _):
            sl = pl.ds(kk * _L, _L)
            mrow_buf[sl] = mwv * me_buf[sl]
            return _
        lax.fori_loop(0, D // _L, scale_me, 0)

        def group(g, _):
            off = (base + g * _GROUP) * D
            pltpu.sync_copy(tok_hbm.at[pl.ds(off, gelems)], tok_buf)
            pltpu.sync_copy(pos_hbm.at[pl.ds(off, gelems)], pos_buf)

            def row(j, _):
                mj = mask_buf[pl.ds((g * _GROUP + j) * _L, _L)]

                def col(kk, _):
                    sl = pl.ds(j * D + kk * _L, _L)
                    msl = pl.ds(kk * _L, _L)
                    tok_buf[sl] = (twv * tok_buf[sl] + pwv * pos_buf[sl]
                                   + mj * mrow_buf[msl])
                    return _
                lax.fori_loop(0, D // _L, col, 0)
                return _
            lax.fori_loop(0, _GROUP, row, 0)

            pltpu.sync_copy(tok_buf, out_hbm.at[pl.ds(off, gelems)])
            return _
        lax.fori_loop(0, ngrp, group, 0)

    return k


def kernel(token_embeds, mask_embeds, position_embeds, mask_inds,
           token_weight, mask_weight, position_weight):
    B, S, D = token_embeds.shape
    N = B * S
    NW = 32

    tok1 = token_embeds.reshape(N * D)
    pos1 = position_embeds.reshape(N * D)
    maskx = jnp.broadcast_to(
        mask_inds.reshape(N, 1).astype(jnp.float32), (N, _L)).reshape(N * _L)
    tw16 = jnp.broadcast_to(token_weight, (_L,))
    pw16 = jnp.broadcast_to(position_weight, (_L,))
    mw16 = jnp.broadcast_to(mask_weight, (_L,))

    out = _sc_mix(N, D, NW)(tok1, pos1, maskx, mask_embeds, tw16, pw16, mw16)
    return out.reshape(B, S, D)


# speedup vs baseline: 2.5671x; 2.5671x over previous
import functools
import jax
import jax.numpy as jnp
from jax import lax
from jax.experimental import pallas as pl
from jax.experimental.pallas import tpu as pltpu
from jax.experimental.pallas import tpu_sc as plsc

_ROWS = 256
_NBUF = 4
_L = 16
_GROUP = 16
_NSC = 1024


def _make_tc_body(NTC, D, R, NBUF):
    nblk = NTC // R

    def body(tok_hbm, pos_hbm, m_ref, me_ref, tw_ref, pw_ref, mw_ref,
             out_hbm, tok_buf, pos_buf, out_buf, sems):
        tw = tw_ref[0]
        pw = pw_ref[0]
        mw = mw_ref[0]
        mrow = mw * me_ref[0, :]

        def in_copies(i, slot):
            return (
                pltpu.make_async_copy(
                    tok_hbm.at[pl.ds(i * R, R), :], tok_buf.at[slot],
                    sems.at[slot, 0]),
                pltpu.make_async_copy(
                    pos_hbm.at[pl.ds(i * R, R), :], pos_buf.at[slot],
                    sems.at[slot, 1]),
            )

        def out_copy(i, slot):
            return pltpu.make_async_copy(
                out_buf.at[slot], out_hbm.at[pl.ds(i * R, R), :],
                sems.at[slot, 2])

        for k in range(min(NBUF - 1, nblk)):
            for c in in_copies(k, k % NBUF):
                c.start()

        def step(i, carry):
            slot = jax.lax.rem(i, NBUF)
            ctok, cpos = in_copies(i, slot)
            ctok.wait()
            cpos.wait()

            @pl.when(i >= NBUF)
            def _():
                out_copy(i - NBUF, slot).wait()

            m = jnp.where(m_ref[i, 0, :], 1.0, 0.0)[:, None]
            out_buf[slot] = (tw * tok_buf[slot] + pw * pos_buf[slot]
                             + m * mrow[None, :])
            out_copy(i, slot).start()

            nxt = i + NBUF - 1
            @pl.when(nxt < nblk)
            def _():
                for c in in_copies(nxt, jax.lax.rem(nxt, NBUF)):
                    c.start()

            return carry

        jax.lax.fori_loop(0, nblk, step, 0)

        tail = min(NBUF, nblk)
        for k in range(tail):
            j = nblk - tail + k
            out_copy(j, j % NBUF).wait()

    return body


def _tc_mix(tok2, pos2, maskb, me2, tw, pw, mw, NTC, D):
    R = _ROWS
    nblk = NTC // R
    return pl.pallas_call(
        _make_tc_body(NTC, D, R, _NBUF),
        in_specs=[
            pl.BlockSpec(memory_space=pltpu.HBM),
            pl.BlockSpec(memory_space=pltpu.HBM),
            pl.BlockSpec(memory_space=pltpu.VMEM),
            pl.BlockSpec(memory_space=pltpu.VMEM),
            pl.BlockSpec(memory_space=pltpu.SMEM),
            pl.BlockSpec(memory_space=pltpu.SMEM),
            pl.BlockSpec(memory_space=pltpu.SMEM),
        ],
        out_specs=pl.BlockSpec(memory_space=pltpu.HBM),
        out_shape=jax.ShapeDtypeStruct((NTC, D), jnp.float32),
        scratch_shapes=[
            pltpu.VMEM((_NBUF, R, D), jnp.float32),
            pltpu.VMEM((_NBUF, R, D), jnp.float32),
            pltpu.VMEM((_NBUF, R, D), jnp.float32),
            pltpu.SemaphoreType.DMA((_NBUF, 3)),
        ],
    )(tok2, pos2, maskb, me2, tw, pw, mw)


def _sc_mix(NSC, D, NW):
    rpw = NSC // NW
    ngrp = rpw // _GROUP
    gelems = _GROUP * D

    mesh = plsc.VectorSubcoreMesh(core_axis_name="c", subcore_axis_name="s")

    @functools.partial(
        pl.kernel,
        out_type=jax.ShapeDtypeStruct((NSC * D,), jnp.float32),
        mesh=mesh,
        scratch_types=[
            pltpu.VMEM((gelems,), jnp.float32),
            pltpu.VMEM((gelems,), jnp.float32),
            pltpu.VMEM((rpw * _L,), jnp.float32),
            pltpu.VMEM((D,), jnp.float32),
            pltpu.VMEM((D,), jnp.float32),
            pltpu.VMEM((_L,), jnp.float32),
            pltpu.VMEM((_L,), jnp.float32),
            pltpu.VMEM((_L,), jnp.float32),
        ],
    )
    def k(tok_hbm, pos_hbm, mask_hbm, me_hbm, tw_hbm, pw_hbm, mw_hbm,
          out_hbm, tok_buf, pos_buf, mask_buf, me_buf, mrow_buf,
          tw_buf, pw_buf, mw_buf):
        wid = lax.axis_index("s") * 2 + lax.axis_index("c")
        base = wid * rpw

        pltpu.sync_copy(tw_hbm, tw_buf)
        pltpu.sync_copy(pw_hbm, pw_buf)
        pltpu.sync_copy(mw_hbm, mw_buf)
        pltpu.sync_copy(me_hbm, me_buf)
        pltpu.sync_copy(mask_hbm.at[pl.ds(base * _L, rpw * _L)], mask_buf)

        twv = tw_buf[...]
        pwv = pw_buf[...]
        mwv = mw_buf[...]

        def scale_me(kk, _):
            sl = pl.ds(kk * _L, _L)
            mrow_buf[sl] = mwv * me_buf[sl]
            return _
        lax.fori_loop(0, D // _L, scale_me, 0)

        def group(g, _):
            off = (base + g * _GROUP) * D
            pltpu.sync_copy(tok_hbm.at[pl.ds(off, gelems)], tok_buf)
            pltpu.sync_copy(pos_hbm.at[pl.ds(off, gelems)], pos_buf)

            def row(j, _):
                mj = mask_buf[pl.ds((g * _GROUP + j) * _L, _L)]

                def col(kk, _):
                    sl = pl.ds(j * D + kk * _L, _L)
                    msl = pl.ds(kk * _L, _L)
                    tok_buf[sl] = (twv * tok_buf[sl] + pwv * pos_buf[sl]
                                   + mj * mrow_buf[msl])
                    return _
                lax.fori_loop(0, D // _L, col, 0)
                return _
            lax.fori_loop(0, _GROUP, row, 0)

            pltpu.sync_copy(tok_buf, out_hbm.at[pl.ds(off, gelems)])
            return _
        lax.fori_loop(0, ngrp, group, 0)

    return k


def kernel(token_embeds, mask_embeds, position_embeds, mask_inds,
           token_weight, mask_weight, position_weight):
    B, S, D = token_embeds.shape
    N = B * S
    NSC = _NSC
    NTC = N - NSC
    nblk = NTC // _ROWS

    tok2 = token_embeds.reshape(N, D)
    pos2 = position_embeds.reshape(N, D)
    maskb = mask_inds.reshape(N)

    tok_tail = tok2[NTC:].reshape(NSC * D)
    pos_tail = pos2[NTC:].reshape(NSC * D)
    maskx = jnp.broadcast_to(
        maskb[NTC:].reshape(NSC, 1).astype(jnp.float32),
        (NSC, _L)).reshape(NSC * _L)
    tw16 = jnp.broadcast_to(token_weight, (_L,))
    pw16 = jnp.broadcast_to(position_weight, (_L,))
    mw16 = jnp.broadcast_to(mask_weight, (_L,))

    me2 = mask_embeds.reshape(1, D)
    maskb_tc = maskb[:NTC].reshape(nblk, 1, _ROWS)

    head = _tc_mix(tok2[:NTC], pos2[:NTC], maskb_tc, me2,
                   token_weight, position_weight, mask_weight, NTC, D)
    tail = _sc_mix(NSC, D, 32)(tok_tail, pos_tail, maskx, mask_embeds,
                               tw16, pw16, mw16)

    out = jnp.concatenate([head, tail.reshape(NSC, D)], axis=0)
    return out.reshape(B, S, D)


# speedup vs baseline: 3.0878x; 1.2029x over previous
import functools
import jax
import jax.numpy as jnp
from jax import lax
from jax.experimental import pallas as pl
from jax.experimental.pallas import tpu as pltpu
from jax.experimental.pallas import tpu_sc as plsc

_ROWS = 256
_NBUF = 4
_L = 16
_GROUP = 16
_NSC = 1024


def _make_tc_body(NTC, D, R, NBUF):
    nblk = NTC // R

    def body(tok_hbm, pos_hbm, m_ref, me_ref, tw_ref, pw_ref, mw_ref,
             out_hbm, tok_buf, pos_buf, out_buf, sems):
        tw = tw_ref[0]
        pw = pw_ref[0]
        mw = mw_ref[0]
        mrow = mw * me_ref[0, :]

        def in_copies(i, slot):
            return (
                pltpu.make_async_copy(
                    tok_hbm.at[pl.ds(i * R, R), :], tok_buf.at[slot],
                    sems.at[slot, 0]),
                pltpu.make_async_copy(
                    pos_hbm.at[pl.ds(i * R, R), :], pos_buf.at[slot],
                    sems.at[slot, 1]),
            )

        def out_copy(i, slot):
            return pltpu.make_async_copy(
                out_buf.at[slot], out_hbm.at[pl.ds(i * R, R), :],
                sems.at[slot, 2])

        for k in range(min(NBUF - 1, nblk)):
            for c in in_copies(k, k % NBUF):
                c.start()

        def step(i, carry):
            slot = jax.lax.rem(i, NBUF)
            ctok, cpos = in_copies(i, slot)
            ctok.wait()
            cpos.wait()

            @pl.when(i >= NBUF)
            def _():
                out_copy(i - NBUF, slot).wait()

            m = jnp.where(m_ref[i, 0, :], 1.0, 0.0)[:, None]
            out_buf[slot] = (tw * tok_buf[slot] + pw * pos_buf[slot]
                             + m * mrow[None, :])
            out_copy(i, slot).start()

            nxt = i + NBUF - 1
            @pl.when(nxt < nblk)
            def _():
                for c in in_copies(nxt, jax.lax.rem(nxt, NBUF)):
                    c.start()

            return carry

        jax.lax.fori_loop(0, nblk, step, 0)

        tail = min(NBUF, nblk)
        for k in range(tail):
            j = nblk - tail + k
            out_copy(j, j % NBUF).wait()

    return body


def _tc_mix(tok2, pos2, maskb, me2, tw, pw, mw, N, NTC, D):
    R = _ROWS
    return pl.pallas_call(
        _make_tc_body(NTC, D, R, _NBUF),
        in_specs=[
            pl.BlockSpec(memory_space=pltpu.HBM),
            pl.BlockSpec(memory_space=pltpu.HBM),
            pl.BlockSpec(memory_space=pltpu.VMEM),
            pl.BlockSpec(memory_space=pltpu.VMEM),
            pl.BlockSpec(memory_space=pltpu.SMEM),
            pl.BlockSpec(memory_space=pltpu.SMEM),
            pl.BlockSpec(memory_space=pltpu.SMEM),
        ],
        out_specs=pl.BlockSpec(memory_space=pltpu.HBM),
        out_shape=jax.ShapeDtypeStruct((N, D), jnp.float32),
        scratch_shapes=[
            pltpu.VMEM((_NBUF, R, D), jnp.float32),
            pltpu.VMEM((_NBUF, R, D), jnp.float32),
            pltpu.VMEM((_NBUF, R, D), jnp.float32),
            pltpu.SemaphoreType.DMA((_NBUF, 3)),
        ],
    )(tok2, pos2, maskb, me2, tw, pw, mw)


def _sc_mix(N, NSC, D, NW):
    rpw = NSC // NW
    ngrp = rpw // _GROUP
    gelems = _GROUP * D
    NTC = N - NSC

    mesh = plsc.VectorSubcoreMesh(core_axis_name="c", subcore_axis_name="s")

    @functools.partial(
        pl.kernel,
        out_type=jax.ShapeDtypeStruct((NSC * D,), jnp.float32),
        mesh=mesh,
        scratch_types=[
            pltpu.VMEM((gelems,), jnp.float32),
            pltpu.VMEM((gelems,), jnp.float32),
            pltpu.VMEM((rpw * _L,), jnp.float32),
            pltpu.VMEM((D,), jnp.float32),
            pltpu.VMEM((D,), jnp.float32),
            pltpu.VMEM((_L,), jnp.float32),
            pltpu.VMEM((_L,), jnp.float32),
            pltpu.VMEM((_L,), jnp.float32),
        ],
    )
    def k(tok_hbm, pos_hbm, mask_hbm, me_hbm, tw_hbm, pw_hbm, mw_hbm,
          out_hbm, tok_buf, pos_buf, mask_buf, me_buf, mrow_buf,
          tw_buf, pw_buf, mw_buf):
        wid = lax.axis_index("s") * 2 + lax.axis_index("c")
        base = wid * rpw

        pltpu.sync_copy(tw_hbm, tw_buf)
        pltpu.sync_copy(pw_hbm, pw_buf)
        pltpu.sync_copy(mw_hbm, mw_buf)
        pltpu.sync_copy(me_hbm, me_buf)
        pltpu.sync_copy(mask_hbm.at[pl.ds(base * _L, rpw * _L)], mask_buf)

        twv = tw_buf[...]
        pwv = pw_buf[...]
        mwv = mw_buf[...]

        def scale_me(kk, _):
            sl = pl.ds(kk * _L, _L)
            mrow_buf[sl] = mwv * me_buf[sl]
            return _
        lax.fori_loop(0, D // _L, scale_me, 0)

        def group(g, _):
            src_off = (NTC + base + g * _GROUP) * D
            dst_off = (base + g * _GROUP) * D
            pltpu.sync_copy(tok_hbm.at[pl.ds(src_off, gelems)], tok_buf)
            pltpu.sync_copy(pos_hbm.at[pl.ds(src_off, gelems)], pos_buf)

            def row(j, _):
                mj = mask_buf[pl.ds((g * _GROUP + j) * _L, _L)]

                def col(kk, _):
                    sl = pl.ds(j * D + kk * _L, _L)
                    msl = pl.ds(kk * _L, _L)
                    tok_buf[sl] = (twv * tok_buf[sl] + pwv * pos_buf[sl]
                                   + mj * mrow_buf[msl])
                    return _
                lax.fori_loop(0, D // _L, col, 0)
                return _
            lax.fori_loop(0, _GROUP, row, 0)

            pltpu.sync_copy(tok_buf, out_hbm.at[pl.ds(dst_off, gelems)])
            return _
        lax.fori_loop(0, ngrp, group, 0)

    return k


def kernel(token_embeds, mask_embeds, position_embeds, mask_inds,
           token_weight, mask_weight, position_weight):
    B, S, D = token_embeds.shape
    N = B * S
    NSC = _NSC
    NTC = N - NSC
    nblk = NTC // _ROWS

    tok2 = token_embeds.reshape(N, D)
    pos2 = position_embeds.reshape(N, D)
    tok1 = token_embeds.reshape(N * D)
    pos1 = position_embeds.reshape(N * D)
    maskb = mask_inds.reshape(N)

    maskb_tc = maskb[:NTC].reshape(nblk, 1, _ROWS)
    maskx = jnp.broadcast_to(
        maskb[NTC:].reshape(NSC, 1).astype(jnp.float32),
        (NSC, _L)).reshape(NSC * _L)
    tw16 = jnp.broadcast_to(token_weight, (_L,))
    pw16 = jnp.broadcast_to(position_weight, (_L,))
    mw16 = jnp.broadcast_to(mask_weight, (_L,))
    me2 = mask_embeds.reshape(1, D)

    head = _tc_mix(tok2, pos2, maskb_tc, me2,
                   token_weight, position_weight, mask_weight, N, NTC, D)
    tail = _sc_mix(N, NSC, D, 32)(tok1, pos1, maskx, mask_embeds,
                                  tw16, pw16, mw16)

    out = lax.dynamic_update_slice(head, tail.reshape(NSC, D), (NTC, 0))
    return out.reshape(B, S, D)


# speedup vs baseline: 8.3232x; 2.6955x over previous
import jax
import jax.numpy as jnp
from jax.experimental import pallas as pl
from jax.experimental.pallas import tpu as pltpu

_ROWS = 256
_NBUF = 4
_SROWS = 64
_NS = 4


def _make_body(N, D):
    R, NBUF, SR, NS = _ROWS, _NBUF, _SROWS, _NS
    edge = NS * SR
    nblk = (N - 2 * edge) // R

    def body(tok_hbm, pos_hbm, m_main, m_edge, me_ref, tw_ref, pw_ref, mw_ref,
             out_hbm, tok_buf, pos_buf, out_buf, stok_buf, spos_buf,
             sout_buf, sems, ssems):
        tw = tw_ref[0]
        pw = pw_ref[0]
        mw = mw_ref[0]
        mrow = mw * me_ref[0, :]

        def in_copies(i, slot):
            return (
                pltpu.make_async_copy(
                    tok_hbm.at[pl.ds(edge + i * R, R), :], tok_buf.at[slot],
                    sems.at[slot, 0]),
                pltpu.make_async_copy(
                    pos_hbm.at[pl.ds(edge + i * R, R), :], pos_buf.at[slot],
                    sems.at[slot, 1]),
            )

        def out_copy(i, slot):
            return pltpu.make_async_copy(
                out_buf.at[slot], out_hbm.at[pl.ds(edge + i * R, R), :],
                sems.at[slot, 2])

        def s_in_copies(row0, slot):
            return (
                pltpu.make_async_copy(
                    tok_hbm.at[pl.ds(row0, SR), :], stok_buf.at[slot],
                    ssems.at[slot, 0]),
                pltpu.make_async_copy(
                    pos_hbm.at[pl.ds(row0, SR), :], spos_buf.at[slot],
                    ssems.at[slot, 1]),
            )

        def s_out_copy(row0, slot):
            return pltpu.make_async_copy(
                sout_buf.at[slot], out_hbm.at[pl.ds(row0, SR), :],
                ssems.at[slot, 2])

        def s_compute(flat_row0, slot):
            blk = flat_row0 // SR
            m = jnp.where(m_edge[blk, 0, :], 1.0, 0.0)[:, None]
            sout_buf[slot] = (tw * stok_buf[slot] + pw * spos_buf[slot]
                              + m * mrow[None, :])

        for k in range(NS):
            for c in s_in_copies(k * SR, k):
                c.start()
        for k in range(min(NBUF - 1, nblk)):
            for c in in_copies(k, k % NBUF):
                c.start()

        for k in range(NS):
            for c in s_in_copies(k * SR, k):
                c.wait()
            s_compute(k * SR, k)
            s_out_copy(k * SR, k).start()

        def step(i, carry):
            slot = jax.lax.rem(i, NBUF)
            ctok, cpos = in_copies(i, slot)
            ctok.wait()
            cpos.wait()

            @pl.when(i >= NBUF)
            def _():
                out_copy(i - NBUF, slot).wait()

            m = jnp.where(m_main[i, 0, :], 1.0, 0.0)[:, None]
            out_buf[slot] = (tw * tok_buf[slot] + pw * pos_buf[slot]
                             + m * mrow[None, :])
            out_copy(i, slot).start()

            nxt = i + NBUF - 1
            @pl.when(nxt < nblk)
            def _():
                for c in in_copies(nxt, jax.lax.rem(nxt, NBUF)):
                    c.start()

            return carry

        jax.lax.fori_loop(0, nblk, step, 0)

        tail0 = N - edge
        for k in range(NS):
            for c in s_in_copies(tail0 + k * SR, k):
                c.start()
        for k in range(NS):
            s_out_copy(k * SR, k).wait()
        for k in range(NS):
            for c in s_in_copies(tail0 + k * SR, k):
                c.wait()
            s_compute(tail0 + k * SR, k)
            s_out_copy(tail0 + k * SR, k).start()

        tailn = min(NBUF, nblk)
        for k in range(tailn):
            j = nblk - tailn + k
            out_copy(j, j % NBUF).wait()
        for k in range(NS):
            s_out_copy(tail0 + k * SR, k).wait()

    return body


def kernel(token_embeds, mask_embeds, position_embeds, mask_inds,
           token_weight, mask_weight, position_weight):
    B, S, D = token_embeds.shape
    N = B * S
    R, SR, NS = _ROWS, _SROWS, _NS
    edge = NS * SR
    nblk = (N - 2 * edge) // R

    tok2 = token_embeds.reshape(N, D)
    pos2 = position_embeds.reshape(N, D)
    maskb = mask_inds.reshape(N)
    mask_main = maskb[edge:N - edge].reshape(nblk, 1, R)
    mask_edge = mask_inds.reshape(N // SR, 1, SR)
    me2 = mask_embeds.reshape(1, D)

    out = pl.pallas_call(
        _make_body(N, D),
        in_specs=[
            pl.BlockSpec(memory_space=pltpu.HBM),
            pl.BlockSpec(memory_space=pltpu.HBM),
            pl.BlockSpec(memory_space=pltpu.VMEM),
            pl.BlockSpec(memory_space=pltpu.VMEM),
            pl.BlockSpec(memory_space=pltpu.VMEM),
            pl.BlockSpec(memory_space=pltpu.SMEM),
            pl.BlockSpec(memory_space=pltpu.SMEM),
            pl.BlockSpec(memory_space=pltpu.SMEM),
        ],
        out_specs=pl.BlockSpec(memory_space=pltpu.HBM),
        out_shape=jax.ShapeDtypeStruct((N, D), jnp.float32),
        scratch_shapes=[
            pltpu.VMEM((_NBUF, R, D), jnp.float32),
            pltpu.VMEM((_NBUF, R, D), jnp.float32),
            pltpu.VMEM((_NBUF, R, D), jnp.float32),
            pltpu.VMEM((NS, SR, D), jnp.float32),
            pltpu.VMEM((NS, SR, D), jnp.float32),
            pltpu.VMEM((NS, SR, D), jnp.float32),
            pltpu.SemaphoreType.DMA((_NBUF, 3)),
            pltpu.SemaphoreType.DMA((NS, 3)),
        ],
    )(tok2, pos2, mask_main, mask_edge, me2,
      token_weight, position_weight, mask_weight)
    return out.reshape(B, S, D)


# speedup vs baseline: 8.5614x; 1.0286x over previous
import jax
import jax.numpy as jnp
from jax.experimental import pallas as pl
from jax.experimental.pallas import tpu as pltpu

_ROWS = 256
_NBUF = 6


def _make_body(N, D, R, NBUF):
    nblk = N // R

    def body(tok_hbm, pos_hbm, m_ref, me_ref, tw_ref, pw_ref, mw_ref,
             out_hbm, tok_buf, pos_buf, out_buf, sems):
        tw = tw_ref[0]
        pw = pw_ref[0]
        mw = mw_ref[0]
        mrow = mw * me_ref[0, :]

        def in_copies(i, slot):
            return (
                pltpu.make_async_copy(
                    tok_hbm.at[pl.ds(i * R, R), :], tok_buf.at[slot],
                    sems.at[slot, 0]),
                pltpu.make_async_copy(
                    pos_hbm.at[pl.ds(i * R, R), :], pos_buf.at[slot],
                    sems.at[slot, 1]),
            )

        def out_copy(i, slot):
            return pltpu.make_async_copy(
                out_buf.at[slot], out_hbm.at[pl.ds(i * R, R), :],
                sems.at[slot, 2])

        for k in range(min(NBUF - 1, nblk)):
            for c in in_copies(k, k % NBUF):
                c.start()

        def step(i, carry):
            slot = jax.lax.rem(i, NBUF)
            ctok, cpos = in_copies(i, slot)
            ctok.wait()
            cpos.wait()

            @pl.when(i >= NBUF)
            def _():
                out_copy(i - NBUF, slot).wait()

            m = jnp.where(m_ref[i, 0, :], 1.0, 0.0)[:, None]
            out_buf[slot] = (tw * tok_buf[slot] + pw * pos_buf[slot]
                             + m * mrow[None, :])
            out_copy(i, slot).start()

            nxt = i + NBUF - 1
            @pl.when(nxt < nblk)
            def _():
                for c in in_copies(nxt, jax.lax.rem(nxt, NBUF)):
                    c.start()

            return carry

        jax.lax.fori_loop(0, nblk, step, 0)

        tail = min(NBUF, nblk)
        for k in range(tail):
            j = nblk - tail + k
            out_copy(j, j % NBUF).wait()

    return body


def kernel(token_embeds, mask_embeds, position_embeds, mask_inds,
           token_weight, mask_weight, position_weight):
    B, S, D = token_embeds.shape
    N = B * S
    R = _ROWS
    nblk = N // R

    tok2 = token_embeds.reshape(N, D)
    pos2 = position_embeds.reshape(N, D)
    maskf = mask_inds.reshape(nblk, 1, R)
    me2 = mask_embeds.reshape(1, D)

    out = pl.pallas_call(
        _make_body(N, D, R, _NBUF),
        in_specs=[
            pl.BlockSpec(memory_space=pltpu.HBM),
            pl.BlockSpec(memory_space=pltpu.HBM),
            pl.BlockSpec(memory_space=pltpu.VMEM),
            pl.BlockSpec(memory_space=pltpu.VMEM),
            pl.BlockSpec(memory_space=pltpu.SMEM),
            pl.BlockSpec(memory_space=pltpu.SMEM),
            pl.BlockSpec(memory_space=pltpu.SMEM),
        ],
        out_specs=pl.BlockSpec(memory_space=pltpu.HBM),
        out_shape=jax.ShapeDtypeStruct((N, D), jnp.float32),
        scratch_shapes=[
            pltpu.VMEM((_NBUF, R, D), jnp.float32),
            pltpu.VMEM((_NBUF, R, D), jnp.float32),
            pltpu.VMEM((_NBUF, R, D), jnp.float32),
            pltpu.SemaphoreType.DMA((_NBUF, 3)),
        ],
    )(tok2, pos2, maskf, me2, token_weight, position_weight, mask_weight)
    return out.reshape(B, S, D)


# speedup vs baseline: 8.5797x; 1.0021x over previous
import jax
import jax.numpy as jnp
from jax.experimental import pallas as pl
from jax.experimental.pallas import tpu as pltpu

_ROWS = 256
_NBUF = 4


def _make_body(N, D, R, NBUF):
    nblk = N // R

    def body(tok_hbm, pos_hbm, m_ref, me_ref, tw_ref, pw_ref, mw_ref,
             out_hbm, tok_buf, pos_buf, out_buf, sems):
        tw = tw_ref[0]
        pw = pw_ref[0]
        mw = mw_ref[0]
        mrow = mw * me_ref[0, :]

        def in_copies(i, slot):
            return (
                pltpu.make_async_copy(
                    tok_hbm.at[pl.ds(i * R, R), :], tok_buf.at[slot],
                    sems.at[slot, 0]),
                pltpu.make_async_copy(
                    pos_hbm.at[pl.ds(i * R, R), :], pos_buf.at[slot],
                    sems.at[slot, 1]),
            )

        def out_copy(i, slot):
            return pltpu.make_async_copy(
                out_buf.at[slot], out_hbm.at[pl.ds(i * R, R), :],
                sems.at[slot, 2])

        for k in range(min(NBUF - 1, nblk)):
            for c in in_copies(k, k % NBUF):
                c.start()

        def step(i, carry):
            slot = jax.lax.rem(i, NBUF)
            ctok, cpos = in_copies(i, slot)
            ctok.wait()
            cpos.wait()

            @pl.when(i >= NBUF)
            def _():
                out_copy(i - NBUF, slot).wait()

            m = jnp.where(m_ref[i, 0, :], 1.0, 0.0)[:, None]
            out_buf[slot] = (tw * tok_buf[slot] + pw * pos_buf[slot]
                             + m * mrow[None, :])
            out_copy(i, slot).start()

            nxt = i + NBUF - 1
            @pl.when(nxt < nblk)
            def _():
                for c in in_copies(nxt, jax.lax.rem(nxt, NBUF)):
                    c.start()

            return carry

        jax.lax.fori_loop(0, nblk, step, 0)

        tail = min(NBUF, nblk)
        for k in range(tail):
            j = nblk - tail + k
            out_copy(j, j % NBUF).wait()

    return body


def kernel(token_embeds, mask_embeds, position_embeds, mask_inds,
           token_weight, mask_weight, position_weight):
    B, S, D = token_embeds.shape
    N = B * S
    R = _ROWS
    nblk = N // R

    tok2 = token_embeds.reshape(N, D)
    pos2 = position_embeds.reshape(N, D)
    maskf = mask_inds.reshape(nblk, 1, R)
    me2 = mask_embeds.reshape(1, D)

    out = pl.pallas_call(
        _make_body(N, D, R, _NBUF),
        in_specs=[
            pl.BlockSpec(memory_space=pltpu.HBM),
            pl.BlockSpec(memory_space=pltpu.HBM),
            pl.BlockSpec(memory_space=pltpu.VMEM),
            pl.BlockSpec(memory_space=pltpu.VMEM),
            pl.BlockSpec(memory_space=pltpu.SMEM),
            pl.BlockSpec(memory_space=pltpu.SMEM),
            pl.BlockSpec(memory_space=pltpu.SMEM),
        ],
        out_specs=pl.BlockSpec(memory_space=pltpu.HBM),
        out_shape=jax.ShapeDtypeStruct((N, D), jnp.float32),
        scratch_shapes=[
            pltpu.VMEM((_NBUF, R, D), jnp.float32),
            pltpu.VMEM((_NBUF, R, D), jnp.float32),
            pltpu.VMEM((_NBUF, R, D), jnp.float32),
            pltpu.SemaphoreType.DMA((_NBUF, 3)),
        ],
    )(tok2, pos2, maskf, me2, token_weight, position_weight, mask_weight)
    return out.reshape(B, S, D)
